# baseline (speedup 1.0000x reference)
import functools

import jax
import jax.numpy as jnp
from jax.experimental import pallas as pl
from jax.experimental.pallas import tpu as pltpu

_BK = 1024


def _coef_kernel(xP_ref, ohT_ref, cfT_ref, out_ref, acc_ref, *, nk, k_total, bk):
    k = pl.program_id(0)

    @pl.when(k == 0)
    def _():
        acc_ref[...] = jnp.zeros_like(acc_ref)

    oh = ohT_ref[...]
    cf = cfT_ref[...]

    @pl.when(k == nk - 1)
    def _():
        rem = k_total - k * bk
        row_o = jax.lax.broadcasted_iota(jnp.int32, oh.shape, 0)
        col_c = jax.lax.broadcasted_iota(jnp.int32, cf.shape, 1)
        oh_m = jnp.where(row_o < rem, oh, 0.0)
        cf_m = jnp.where(col_c < rem, cf, 0.0)
        acc_ref[...] += jnp.dot(cf_m, oh_m, preferred_element_type=jnp.float32)

    @pl.when(k < nk - 1)
    def _():
        acc_ref[...] += jnp.dot(cf, oh, preferred_element_type=jnp.float32)

    @pl.when(k == nk - 1)
    def _():
        xv = xP_ref[...]
        acc = acc_ref[...]
        out_ref[...] = jnp.sum(xv * acc[None, :, :], axis=1)


def kernel(x, user_onehot, coef):
    num_trips, num_items, num_params = x.shape
    k_total = user_onehot.shape[1]

    ohT = user_onehot.T
    cfT = coef.T
    xP = jnp.transpose(x, (1, 2, 0))

    nk = pl.cdiv(k_total, _BK)

    out26 = pl.pallas_call(
        functools.partial(_coef_kernel, nk=nk, k_total=k_total, bk=_BK),
        grid=(nk,),
        in_specs=[
            pl.BlockSpec((num_items, num_params, num_trips), lambda k: (0, 0, 0)),
            pl.BlockSpec((_BK, num_trips), lambda k: (k, 0)),
            pl.BlockSpec((num_params, _BK), lambda k: (0, k)),
        ],
        out_specs=pl.BlockSpec((num_items, num_trips), lambda k: (0, 0)),
        out_shape=jax.ShapeDtypeStruct((num_items, num_trips), jnp.float32),
        scratch_shapes=[pltpu.VMEM((num_params, num_trips), jnp.float32)],
        compiler_params=pltpu.CompilerParams(
            dimension_semantics=("arbitrary",),
        ),
    )(xP, ohT, cfT)
    return out26.T


# speedup vs baseline: 1.1671x; 1.1671x over previous
import functools

import jax
import jax.numpy as jnp
from jax.experimental import pallas as pl
from jax.experimental.pallas import tpu as pltpu

_BK = 3072


def _coef_kernel(xP_ref, ohT_ref, cfT_ref, out_ref, acc_ref, *, nk, k_total, bk):
    k = pl.program_id(0)

    @pl.when(k == 0)
    def _():
        acc_ref[...] = jnp.zeros_like(acc_ref)

    oh = ohT_ref[...]
    cf = cfT_ref[...]

    @pl.when(k == nk - 1)
    def _():
        rem = k_total - k * bk
        row_o = jax.lax.broadcasted_iota(jnp.int32, oh.shape, 0)
        col_c = jax.lax.broadcasted_iota(jnp.int32, cf.shape, 1)
        oh_m = jnp.where(row_o < rem, oh, 0.0)
        cf_m = jnp.where(col_c < rem, cf, 0.0)
        acc_ref[...] += jnp.dot(cf_m, oh_m, preferred_element_type=jnp.float32)

    @pl.when(k < nk - 1)
    def _():
        acc_ref[...] += jnp.dot(cf, oh, preferred_element_type=jnp.float32)

    @pl.when(k == nk - 1)
    def _():
        xv = xP_ref[...]
        acc = acc_ref[...]
        out_ref[...] = jnp.sum(xv * acc[None, :, :], axis=1)


def kernel(x, user_onehot, coef):
    num_trips, num_items, num_params = x.shape
    k_total = user_onehot.shape[1]

    ohT = user_onehot.T
    cfT = coef.T
    xP = jnp.transpose(x, (1, 2, 0))

    nk = pl.cdiv(k_total, _BK)

    out26 = pl.pallas_call(
        functools.partial(_coef_kernel, nk=nk, k_total=k_total, bk=_BK),
        grid=(nk,),
        in_specs=[
            pl.BlockSpec((num_items, num_params, num_trips), lambda k: (0, 0, 0)),
            pl.BlockSpec((_BK, num_trips), lambda k: (k, 0)),
            pl.BlockSpec((num_params, _BK), lambda k: (0, k)),
        ],
        out_specs=pl.BlockSpec((num_items, num_trips), lambda k: (0, 0)),
        out_shape=jax.ShapeDtypeStruct((num_items, num_trips), jnp.float32),
        scratch_shapes=[pltpu.VMEM((num_params, num_trips), jnp.float32)],
        compiler_params=pltpu.CompilerParams(
            dimension_semantics=("arbitrary",),
        ),
    )(xP, ohT, cfT)
    return out26.T


# speedup vs baseline: 1.2386x; 1.0613x over previous
import functools

import jax
import jax.numpy as jnp
from jax.experimental import pallas as pl
from jax.experimental.pallas import tpu as pltpu

_BK = 2048


def _coef_kernel(xP_ref, ohT_ref, cfT_ref, out_ref, acc_ref, *, nk, rem):
    k = pl.program_id(0)

    @pl.when(k == 0)
    def _():
        acc_ref[...] = jnp.zeros_like(acc_ref)

    @pl.when(k < nk - 1)
    def _():
        acc_ref[...] += jnp.dot(
            cfT_ref[...], ohT_ref[...], preferred_element_type=jnp.float32)

    @pl.when(k == nk - 1)
    def _():
        acc_ref[...] += jnp.dot(
            cfT_ref[:, :rem], ohT_ref[:rem, :],
            preferred_element_type=jnp.float32)
        xv = xP_ref[...]
        acc = acc_ref[...]
        out_ref[...] = jnp.sum(xv * acc[None, :, :], axis=1)


def kernel(x, user_onehot, coef):
    num_trips, num_items, num_params = x.shape
    k_total = user_onehot.shape[1]

    ohT = user_onehot.T
    cfT = coef.T
    xP = jnp.transpose(x, (1, 2, 0))

    nk = pl.cdiv(k_total, _BK)
    rem = k_total - (nk - 1) * _BK

    out26 = pl.pallas_call(
        functools.partial(_coef_kernel, nk=nk, rem=rem),
        grid=(nk,),
        in_specs=[
            pl.BlockSpec((num_items, num_params, num_trips), lambda k: (0, 0, 0)),
            pl.BlockSpec((_BK, num_trips), lambda k: (k, 0)),
            pl.BlockSpec((num_params, _BK), lambda k: (0, k)),
        ],
        out_specs=pl.BlockSpec((num_items, num_trips), lambda k: (0, 0)),
        out_shape=jax.ShapeDtypeStruct((num_items, num_trips), jnp.float32),
        scratch_shapes=[pltpu.VMEM((num_params, num_trips), jnp.float32)],
        compiler_params=pltpu.CompilerParams(
            dimension_semantics=("arbitrary",),
        ),
    )(xP, ohT, cfT)
    return out26.T
